# baseline (speedup 1.0000x reference)
import functools

import jax
import jax.numpy as jnp
from jax import lax
from jax.experimental import pallas as pl
from jax.experimental.pallas import tpu as pltpu
from jax.experimental.pallas import tpu_sc as plsc

_N = 10000
_K = 320000
_RANK = 566
_NTILES = 32
_CHUNK = 320
_SLOTS = 80
_CAND = _NTILES * _SLOTS
_TR = 400


def _prepare_kernel(p_ref, out_ref, cut_ref):
    p = p_ref[...]
    pmax = jnp.max(p)
    cnt_max = jnp.sum((p == pmax).astype(jnp.int32))
    m2 = jnp.max(jnp.where(p < pmax, p, -1.0))
    ub = pmax * jnp.where(cnt_max >= 2, pmax, m2)

    def cond(carry):
        lo, hi = carry
        return hi - lo > 1

    def body(carry):
        lo, hi = carry
        mid = (lo + hi) // 2
        t = lax.bitcast_convert_type(mid, jnp.float32)
        ge = jnp.sum((p >= t).astype(jnp.int32)) >= _RANK + 1
        return jnp.where(ge, mid, lo), jnp.where(ge, hi, mid)

    hi0 = lax.bitcast_convert_type(pmax, jnp.int32) + 1
    lo, _ = lax.while_loop(cond, body, (jnp.int32(0), hi0))
    l567 = lax.bitcast_convert_type(lo, jnp.float32)
    lb = l567 * l567
    cutoff = (lb / pmax) * (1.0 - 3e-7)

    lane = lax.broadcasted_iota(jnp.int32, (1, 16), 1)
    vals = jnp.where(lane == 0, cutoff,
                     jnp.where(lane == 1, lb,
                               jnp.where(lane == 2, ub, 0.0)))
    out_ref[...] = vals
    cut_ref[...] = jnp.broadcast_to(cutoff, (1, 16))


def _sc_compact_body(p_hbm, cut_hbm, out_hbm, chunk_v, cut_v, buf_v, idx_v,
                     sem):
    wid = lax.axis_index("s") * 2 + lax.axis_index("c")
    pltpu.sync_copy(p_hbm.at[pl.ds(wid * _CHUNK, _CHUNK)], chunk_v)
    pltpu.sync_copy(cut_hbm, cut_v)

    cut = cut_v[...]

    zero = jnp.zeros((16,), jnp.float32)
    for i in range(_SLOTS // 16):
        buf_v[pl.ds(i * 16, 16)] = zero
    pltpu.sync_copy(buf_v, out_hbm.at[pl.ds(wid * _SLOTS, _SLOTS)])

    one = jnp.ones((16,), jnp.int32)
    nil = jnp.zeros((16,), jnp.int32)
    off = jnp.full((16,), wid * _SLOTS, jnp.int32)
    lanes = lax.iota(jnp.int32, 16)
    last = jnp.full((16,), 15, jnp.int32)
    trash = jnp.full((16,), _CAND, jnp.int32) + lanes
    for i in range(_CHUNK // 16):
        v = chunk_v[pl.ds(i * 16, 16)]
        m = v >= cut
        x = jnp.where(m, one, nil)
        for sh in (1, 2, 4, 8):
            shifted = x.at[jnp.maximum(lanes - sh, nil)].get(
                mode="promise_in_bounds")
            x = x + jnp.where(lanes >= sh, shifted, nil)
        pos = x - one + off
        idx_v[pl.ds(i * 16, 16)] = jnp.where(m, pos, trash)
        off = off + x.at[last].get(mode="promise_in_bounds")

    pltpu.async_copy(chunk_v, out_hbm.at[idx_v], sem).wait()


def _threshold_kernel(scal_ref, qrow_ref, qcol_ref, tau_ref):
    qrow = qrow_ref[...]
    qcol = qcol_ref[...]
    prod = qcol * qrow
    diag = qrow * qrow

    lo0 = lax.bitcast_convert_type(scal_ref[0, 1], jnp.int32)
    hi0 = lax.bitcast_convert_type(scal_ref[0, 2], jnp.int32) + 1

    def cond(carry):
        lo, hi = carry
        return hi - lo > 1

    def body(carry):
        lo, hi = carry
        mid = (lo + hi) // 2
        t = lax.bitcast_convert_type(mid, jnp.float32)
        c = (jnp.sum((prod >= t).astype(jnp.int32))
             - jnp.sum((diag >= t).astype(jnp.int32)))
        ge = c >= _K
        return jnp.where(ge, mid, lo), jnp.where(ge, hi, mid)

    lo, _ = lax.while_loop(cond, body, (lo0, hi0))
    tau_ref[...] = jnp.broadcast_to(
        lax.bitcast_convert_type(lo, jnp.float32), (1, 1))


def _write_kernel(tau_ref, pcol_ref, prow_ref, out_ref):
    i0 = pl.program_id(0) * _TR
    rows = pcol_ref[...]
    cols = prow_ref[...]
    tau = tau_ref[0, 0]
    prod = rows * cols
    ridx = lax.broadcasted_iota(jnp.int32, (_TR, _N), 0) + i0
    cidx = lax.broadcasted_iota(jnp.int32, (_TR, _N), 1)
    keep = (prod >= tau) & (ridx != cidx)
    out_ref[...] = keep.astype(jnp.float32)


def kernel(pred):
    p = pred.reshape(-1)

    scal, cut = pl.pallas_call(
        _prepare_kernel,
        out_shape=(jax.ShapeDtypeStruct((1, 16), jnp.float32),
                   jax.ShapeDtypeStruct((1, 16), jnp.float32)),
    )(pred)

    p_pad = jnp.pad(p, (0, _NTILES * _CHUNK - _N))

    sc_compact = functools.partial(
        pl.kernel,
        mesh=plsc.VectorSubcoreMesh(core_axis_name="c", subcore_axis_name="s"),
        out_type=jax.ShapeDtypeStruct((_CAND + 16,), jnp.float32),
        scratch_types=[
            pltpu.VMEM((_CHUNK,), jnp.float32),
            pltpu.VMEM((16,), jnp.float32),
            pltpu.VMEM((_SLOTS,), jnp.float32),
            pltpu.VMEM((_CHUNK,), jnp.int32),
            pltpu.SemaphoreType.DMA,
        ],
    )(_sc_compact_body)
    cands = sc_compact(p_pad, cut.reshape(16))[:_CAND]

    tau = pl.pallas_call(
        _threshold_kernel,
        out_shape=jax.ShapeDtypeStruct((1, 1), jnp.float32),
    )(scal, cands.reshape(1, _CAND), cands.reshape(_CAND, 1))

    out = pl.pallas_call(
        _write_kernel,
        grid=(_N // _TR,),
        in_specs=[
            pl.BlockSpec((1, 1), lambda i: (0, 0)),
            pl.BlockSpec((_TR, 1), lambda i: (i, 0)),
            pl.BlockSpec((1, _N), lambda i: (0, 0)),
        ],
        out_specs=pl.BlockSpec((_TR, _N), lambda i: (i, 0)),
        out_shape=jax.ShapeDtypeStruct((_N, _N), jnp.float32),
        compiler_params=pltpu.CompilerParams(
            dimension_semantics=("parallel",)),
    )(tau, p.reshape(_N, 1), p.reshape(1, _N))
    return out


# speedup vs baseline: 8.2508x; 8.2508x over previous
import jax
import jax.numpy as jnp
import numpy as np
from jax.experimental import pallas as pl
from jax.experimental.pallas import tpu as pltpu

_N = 10000
_K = 320000
_CAND = 1536
_RANK = 566
_TR = 400


def _threshold_kernel(qrow_ref, qcol_ref, tau_ref):
    qrow = qrow_ref[...]
    qcol = qcol_ref[...]
    prod = qcol * qrow
    diag = qrow * qrow

    lb = qrow_ref[0, _RANK] * qrow_ref[0, _RANK]
    ub = qrow_ref[0, 0] * qrow_ref[0, 1]
    lo0 = jax.lax.bitcast_convert_type(lb, jnp.int32)
    hi0 = jax.lax.bitcast_convert_type(ub, jnp.int32) + 1

    def cond(carry):
        lo, hi = carry
        return hi - lo > 1

    def body(carry):
        lo, hi = carry
        mid = (lo + hi) // 2
        t = jax.lax.bitcast_convert_type(mid, jnp.float32)
        c = (jnp.sum((prod >= t).astype(jnp.int32))
             - jnp.sum((diag >= t).astype(jnp.int32)))
        ge = c >= _K
        return jnp.where(ge, mid, lo), jnp.where(ge, hi, mid)

    lo, _ = jax.lax.while_loop(cond, body, (lo0, hi0))
    tau_ref[...] = jnp.broadcast_to(
        jax.lax.bitcast_convert_type(lo, jnp.float32), (1, 1))


def _write_kernel(tau_ref, pcol_ref, prow_ref, out_ref):
    i0 = pl.program_id(0) * _TR
    rows = pcol_ref[...]
    cols = prow_ref[...]
    tau = tau_ref[0, 0]
    prod = rows * cols
    ridx = jax.lax.broadcasted_iota(jnp.int32, (_TR, _N), 0) + i0
    cidx = jax.lax.broadcasted_iota(jnp.int32, (_TR, _N), 1)
    keep = (prod >= tau) & (ridx != cidx)
    out_ref[...] = keep.astype(jnp.float32)


def kernel(pred):
    p = pred.reshape(-1)
    q = jax.lax.top_k(p, _CAND)[0]

    tau = pl.pallas_call(
        _threshold_kernel,
        out_shape=jax.ShapeDtypeStruct((1, 1), jnp.float32),
    )(q.reshape(1, _CAND), q.reshape(_CAND, 1))

    out = pl.pallas_call(
        _write_kernel,
        grid=(_N // _TR,),
        in_specs=[
            pl.BlockSpec((1, 1), lambda i: (0, 0)),
            pl.BlockSpec((_TR, 1), lambda i: (i, 0)),
            pl.BlockSpec((1, _N), lambda i: (0, 0)),
        ],
        out_specs=pl.BlockSpec((_TR, _N), lambda i: (i, 0)),
        out_shape=jax.ShapeDtypeStruct((_N, _N), jnp.float32),
        compiler_params=pltpu.CompilerParams(
            dimension_semantics=("parallel",)),
    )(tau, p.reshape(_N, 1), p.reshape(1, _N))
    return out
